# Initial kernel scaffold; baseline (speedup 1.0000x reference)
import jax
import jax.numpy as jnp
from jax.experimental import pallas as pl


def kernel(h, edge_index, W1, W2, bn1_g, bn1_b, bn2_g, bn2_b, Wp, bp):
    raise NotImplementedError("write your pallas kernel here")


# speedup vs baseline: 6.1005x; 6.1005x over previous
import functools

import jax
import jax.numpy as jnp
from jax import lax
from jax.experimental import pallas as pl
from jax.experimental.pallas import tpu as pltpu
from jax.experimental.pallas import tpu_sc as plsc

NC = 2
NS = 16
CHUNK = 128
SUB = 128


def _sc_agg_build(np_, d, nchunks):
    cpc = nchunks // NC
    rpt = np_ // NS
    nsub = rpt // SUB

    def body(x_hbm, src_hbm, dst_hbm, out_hbm, src_v, dst_v, rows_v, stage_v,
             acc, sem):
        c = lax.axis_index("c")
        s = lax.axis_index("s")

        row0 = s * rpt
        for k in range(nsub):
            r = row0 + k * SUB
            pltpu.sync_copy(x_hbm.at[pl.ds(r, SUB)], stage_v)
            pltpu.sync_copy(stage_v, acc.at[pl.ds(r, SUB)])
        plsc.subcore_barrier()

        lo = c * cpc + (s * cpc) // NS
        hi = c * cpc + ((s + 1) * cpc) // NS

        def step(j, carry):
            pltpu.sync_copy(src_hbm.at[pl.ds(j * CHUNK, CHUNK)], src_v)
            pltpu.sync_copy(dst_hbm.at[pl.ds(j * CHUNK, CHUNK)], dst_v)
            pltpu.async_copy(x_hbm.at[src_v], rows_v, sem).wait()
            pltpu.sync_copy(rows_v, acc.at[dst_v], add=True)
            return carry

        lax.fori_loop(lo, hi, step, 0)
        plsc.subcore_barrier()

        for k in range(nsub):
            r = row0 + k * SUB
            pltpu.sync_copy(acc.at[pl.ds(r, SUB)], stage_v)
            pltpu.sync_copy(stage_v, out_hbm.at[c, pl.ds(r, SUB)])

    mesh = plsc.VectorSubcoreMesh(core_axis_name="c", subcore_axis_name="s")
    return pl.kernel(
        body,
        out_type=jax.ShapeDtypeStruct((NC, np_, d), jnp.float32),
        mesh=mesh,
        scratch_types=[
            pltpu.VMEM((CHUNK,), jnp.int32),
            pltpu.VMEM((CHUNK,), jnp.int32),
            pltpu.VMEM((CHUNK, d), jnp.float32),
            pltpu.VMEM((SUB, d), jnp.float32),
            pltpu.VMEM_SHARED((np_, d), jnp.float32),
            pltpu.SemaphoreType.DMA,
        ],
    )


def _mlp_body(n, parts_ref, x_ref, w1_ref, w2_ref, g1_ref, b1_ref, g2_ref,
              b2_ref, t_ref, pooled_ref):
    np_ = x_ref.shape[0]
    inv_n = 1.0 / n
    mask = lax.broadcasted_iota(jnp.int32, (np_, 1), 0) < n
    z = parts_ref[0] + parts_ref[1] - x_ref[...]
    y = lax.dot_general(z, w1_ref[...], (((1,), (1,)), ((), ())),
                        preferred_element_type=jnp.float32)
    m = jnp.sum(y, axis=0, keepdims=True) * inv_n
    s2 = jnp.sum(y * y, axis=0, keepdims=True) * inv_n
    v = s2 - m * m
    t1 = g1_ref[...] * (y - m) * lax.rsqrt(v + 1e-5) + b1_ref[...]
    t1 = jnp.where(mask, jnp.maximum(t1, 0.0), 0.0)
    u = lax.dot_general(t1, w2_ref[...], (((1,), (1,)), ((), ())),
                        preferred_element_type=jnp.float32)
    m2 = jnp.sum(u, axis=0, keepdims=True) * inv_n
    s22 = jnp.sum(u * u, axis=0, keepdims=True) * inv_n
    v2 = s22 - m2 * m2
    t = g2_ref[...] * (u - m2) * lax.rsqrt(v2 + 1e-5) + b2_ref[...]
    t = jnp.where(mask, jnp.maximum(t, 0.0), 0.0)
    t_ref[...] = t
    pooled_ref[...] = jnp.sum(t, axis=0, keepdims=True)


def _head_body(h_ref, pooled_ref, wp_ref, bp_ref, out_ref):
    p0 = jnp.sum(h_ref[...], axis=0, keepdims=True)
    nl = wp_ref.shape[0]
    acc = lax.dot_general(p0, wp_ref[0], (((1,), (1,)), ((), ())),
                          preferred_element_type=jnp.float32)
    for i in range(1, nl):
        pi = pooled_ref[i - 1].reshape(1, -1)
        acc = acc + lax.dot_general(pi, wp_ref[i], (((1,), (1,)), ((), ())),
                                    preferred_element_type=jnp.float32)
    acc = acc + jnp.sum(bp_ref[...], axis=0, keepdims=True)
    out_ref[...] = acc


def kernel(h, edge_index, W1, W2, bn1_g, bn1_b, bn2_g, bn2_b, Wp, bp):
    n, d = h.shape
    e = edge_index.shape[1]
    nchunks = e // CHUNK
    align = NS * SUB
    np_ = ((n + align - 1) // align) * align
    src = edge_index[0].astype(jnp.int32)
    dst = edge_index[1].astype(jnp.int32)
    h_pad = jnp.pad(h, ((0, np_ - n), (0, 0)))

    sc_agg = _sc_agg_build(np_, d, nchunks)

    num_layers = W1.shape[0]
    mlp = pl.pallas_call(
        functools.partial(_mlp_body, n),
        out_shape=[
            jax.ShapeDtypeStruct((np_, d), jnp.float32),
            jax.ShapeDtypeStruct((1, d), jnp.float32),
        ],
    )

    x = h_pad
    pooled_list = []
    for i in range(num_layers):
        parts = sc_agg(x, src, dst)
        x, pooled = mlp(parts, x, W1[i], W2[i],
                        bn1_g[i].reshape(1, d), bn1_b[i].reshape(1, d),
                        bn2_g[i].reshape(1, d), bn2_b[i].reshape(1, d))
        pooled_list.append(pooled)

    pooled_all = jnp.concatenate(pooled_list, axis=0)
    head = pl.pallas_call(
        _head_body,
        out_shape=jax.ShapeDtypeStruct((1, d), jnp.float32),
    )
    return head(h_pad, pooled_all, Wp, bp)
